# Initial kernel scaffold; baseline (speedup 1.0000x reference)
import jax
import jax.numpy as jnp
from jax.experimental import pallas as pl


def kernel(x, edge_index, W0, b0, g0, be0, W1, b1, g1, be1, W2, b2, g2, be2):
    raise NotImplementedError("write your pallas kernel here")


# speedup vs baseline: 9.8359x; 9.8359x over previous
import functools

import jax
import jax.numpy as jnp
from jax import lax
from jax.experimental import pallas as pl
from jax.experimental.pallas import tpu as pltpu
from jax.experimental.pallas import tpu_sc as plsc

N = 10000
E = 320000
H = 128
LANES = 16
NC = 2
NS = 16
NW = NC * NS
EPT = E // NW
K = 80
NCH = EPT // K
NP = 10240
RPT = NP // NS
ZR = 128
BN_EPS = 1e-5
BM = 2000


def _sc_mesh():
    return plsc.VectorSubcoreMesh(core_axis_name="c", subcore_axis_name="s")


def _deg_sc(dst):
    @functools.partial(
        pl.kernel,
        out_type=jax.ShapeDtypeStruct((NC, NP, H), jnp.float32),
        mesh=_sc_mesh(),
        scratch_types=[
            pltpu.VMEM_SHARED((NP, H), jnp.float32),
            pltpu.VMEM((K,), jnp.int32),
            pltpu.VMEM((K, H), jnp.float32),
            pltpu.VMEM((ZR, H), jnp.float32),
        ],
    )
    def deg_kernel(dst_hbm, out_hbm, accd, dstv, ones_v, zbuf):
        c = lax.axis_index("c")
        s = lax.axis_index("s")
        w = c * NS + s

        def fill_z(i, carry):
            for j in range(H // LANES):
                zbuf[i, pl.ds(j * LANES, LANES)] = jnp.zeros((LANES,), jnp.float32)
            return carry

        lax.fori_loop(0, ZR, fill_z, 0)

        def fill_o(i, carry):
            for j in range(H // LANES):
                ones_v[i, pl.ds(j * LANES, LANES)] = jnp.ones((LANES,), jnp.float32)
            return carry

        lax.fori_loop(0, K, fill_o, 0)

        for kblk in range(RPT // ZR):
            pltpu.sync_copy(zbuf, accd.at[pl.ds(s * RPT + kblk * ZR, ZR)])
        plsc.subcore_barrier()

        def chunk(i, carry):
            base = w * EPT + i * K
            pltpu.sync_copy(dst_hbm.at[pl.ds(base, K)], dstv)
            pltpu.sync_copy(ones_v, accd.at[dstv], add=True)
            return carry

        lax.fori_loop(0, NCH, chunk, 0)

        plsc.subcore_barrier()
        for kblk in range(RPT // ZR):
            pltpu.sync_copy(accd.at[pl.ds(s * RPT + kblk * ZR, ZR)], zbuf)
            pltpu.sync_copy(zbuf, out_hbm.at[c, pl.ds(s * RPT + kblk * ZR, ZR)])

    return deg_kernel(dst)


def _scatter_sc(hs, src, dst):
    @functools.partial(
        pl.kernel,
        out_type=jax.ShapeDtypeStruct((NC, NP, H), jnp.float32),
        mesh=_sc_mesh(),
        scratch_types=[
            pltpu.VMEM_SHARED((NP, H), jnp.float32),
            pltpu.VMEM((K,), jnp.int32),
            pltpu.VMEM((K,), jnp.int32),
            pltpu.VMEM((K, H), jnp.float32),
            pltpu.VMEM((ZR, H), jnp.float32),
            pltpu.SemaphoreType.DMA,
        ],
    )
    def s_kernel(hs_hbm, src_hbm, dst_hbm, out_hbm, acc, srcv, dstv, rows, zbuf, sem):
        c = lax.axis_index("c")
        s = lax.axis_index("s")
        w = c * NS + s

        def fill_z(i, carry):
            for j in range(H // LANES):
                zbuf[i, pl.ds(j * LANES, LANES)] = jnp.zeros((LANES,), jnp.float32)
            return carry

        lax.fori_loop(0, ZR, fill_z, 0)

        for kblk in range(RPT // ZR):
            pltpu.sync_copy(zbuf, acc.at[pl.ds(s * RPT + kblk * ZR, ZR)])
        plsc.subcore_barrier()

        def chunk(i, carry):
            base = w * EPT + i * K
            pltpu.sync_copy(src_hbm.at[pl.ds(base, K)], srcv)
            pltpu.sync_copy(dst_hbm.at[pl.ds(base, K)], dstv)
            pltpu.async_copy(hs_hbm.at[srcv], rows, sem).wait()
            pltpu.sync_copy(rows, acc.at[dstv], add=True)
            return carry

        lax.fori_loop(0, NCH, chunk, 0)

        plsc.subcore_barrier()
        for kblk in range(RPT // ZR):
            pltpu.sync_copy(acc.at[pl.ds(s * RPT + kblk * ZR, ZR)], zbuf)
            pltpu.sync_copy(zbuf, out_hbm.at[c, pl.ds(s * RPT + kblk * ZR, ZR)])

    return s_kernel(hs, src, dst)


def _dinv_tc(degp):
    def body(deg_ref, o_ref):
        d = deg_ref[0, :, 0:1] + deg_ref[1, :, 0:1] + 1.0
        o_ref[...] = jnp.broadcast_to(lax.rsqrt(d), (BM, H))

    return pl.pallas_call(
        body,
        grid=(N // BM,),
        in_specs=[pl.BlockSpec((NC, BM, H), lambda i: (0, i, 0))],
        out_specs=pl.BlockSpec((BM, H), lambda i: (i, 0)),
        out_shape=jax.ShapeDtypeStruct((N, H), jnp.float32),
    )(degp)


def _mm_tc(x, W, dinv):
    def body(x_ref, w_ref, dinv_ref, o_ref):
        h = jnp.dot(x_ref[...], w_ref[...],
                    preferred_element_type=jnp.float32,
                    precision=lax.Precision.HIGHEST)
        o_ref[...] = h * dinv_ref[...]

    return pl.pallas_call(
        body,
        grid=(N // BM,),
        in_specs=[
            pl.BlockSpec((BM, H), lambda i: (i, 0)),
            pl.BlockSpec((H, H), lambda i: (0, 0)),
            pl.BlockSpec((BM, H), lambda i: (i, 0)),
        ],
        out_specs=pl.BlockSpec((BM, H), lambda i: (i, 0)),
        out_shape=jax.ShapeDtypeStruct((N, H), jnp.float32),
    )(x, W, dinv)


def _f1_tc(p, hs, dinv, b):
    def body(p_ref, hs_ref, dinv_ref, b_ref, t_ref, st_ref):
        i = pl.program_id(0)
        t = (p_ref[0] + p_ref[1] + hs_ref[...]) * dinv_ref[...] + b_ref[...]
        t_ref[...] = t

        @pl.when(i == 0)
        def _():
            st_ref[...] = jnp.zeros((8, H), jnp.float32)

        st_ref[0:1, :] += jnp.sum(t, axis=0, keepdims=True)
        st_ref[1:2, :] += jnp.sum(t * t, axis=0, keepdims=True)

    return pl.pallas_call(
        body,
        grid=(N // BM,),
        in_specs=[
            pl.BlockSpec((NC, BM, H), lambda i: (0, i, 0)),
            pl.BlockSpec((BM, H), lambda i: (i, 0)),
            pl.BlockSpec((BM, H), lambda i: (i, 0)),
            pl.BlockSpec((1, H), lambda i: (0, 0)),
        ],
        out_specs=[
            pl.BlockSpec((BM, H), lambda i: (i, 0)),
            pl.BlockSpec((8, H), lambda i: (0, 0)),
        ],
        out_shape=[
            jax.ShapeDtypeStruct((N, H), jnp.float32),
            jax.ShapeDtypeStruct((8, H), jnp.float32),
        ],
    )(p, hs, dinv, b)


def _f2_tc(t, stats, g, be, x):
    def body(t_ref, st_ref, g_ref, be_ref, x_ref, o_ref):
        m = st_ref[0:1, :] * (1.0 / N)
        ex2 = st_ref[1:2, :] * (1.0 / N)
        v = ex2 - m * m
        scale = lax.rsqrt(v + BN_EPS) * g_ref[...]
        h = (t_ref[...] - m) * scale + be_ref[...]
        o_ref[...] = x_ref[...] + jnp.maximum(h, 0.0)

    return pl.pallas_call(
        body,
        grid=(N // BM,),
        in_specs=[
            pl.BlockSpec((BM, H), lambda i: (i, 0)),
            pl.BlockSpec((8, H), lambda i: (0, 0)),
            pl.BlockSpec((1, H), lambda i: (0, 0)),
            pl.BlockSpec((1, H), lambda i: (0, 0)),
            pl.BlockSpec((BM, H), lambda i: (i, 0)),
        ],
        out_specs=pl.BlockSpec((BM, H), lambda i: (i, 0)),
        out_shape=jax.ShapeDtypeStruct((N, H), jnp.float32),
    )(t, stats, g, be, x)


def kernel(x, edge_index, W0, b0, g0, be0, W1, b1, g1, be1, W2, b2, g2, be2):
    src = edge_index[0]
    dst = edge_index[1]

    degp = _deg_sc(dst)
    dinv = _dinv_tc(degp)

    for (W, b, g, be) in ((W0, b0, g0, be0), (W1, b1, g1, be1), (W2, b2, g2, be2)):
        hs = _mm_tc(x, W, dinv)
        p = _scatter_sc(hs, src, dst)
        t, stats = _f1_tc(p, hs, dinv, b.reshape(1, H))
        x = _f2_tc(t, stats, g.reshape(1, H), be.reshape(1, H), x)
    return x


# speedup vs baseline: 12.8819x; 1.3097x over previous
import functools

import jax
import jax.numpy as jnp
from jax import lax
from jax.experimental import pallas as pl
from jax.experimental.pallas import tpu as pltpu
from jax.experimental.pallas import tpu_sc as plsc

N = 10000
E = 320000
H = 128
LANES = 16
NC = 2
NS = 16
NW = NC * NS
EPT = E // NW
K = 40
NCH = EPT // K
NP = 10240
RPT = NP // NS
ZR = 64
BN_EPS = 1e-5
BM = 2000


def _sc_mesh():
    return plsc.VectorSubcoreMesh(core_axis_name="c", subcore_axis_name="s")


def _deg_sc(dst):
    @functools.partial(
        pl.kernel,
        out_type=jax.ShapeDtypeStruct((NC, NP, H), jnp.float32),
        mesh=_sc_mesh(),
        scratch_types=[
            pltpu.VMEM_SHARED((NP, H), jnp.float32),
            pltpu.VMEM((NCH, K), jnp.int32),
            pltpu.VMEM((K, H), jnp.float32),
            pltpu.VMEM((ZR, H), jnp.float32),
        ],
    )
    def deg_kernel(dst_hbm, out_hbm, accd, dstp, ones_v, zbuf):
        c = lax.axis_index("c")
        s = lax.axis_index("s")
        w = c * NS + s

        def fill_z(i, carry):
            for j in range(H // LANES):
                zbuf[i, pl.ds(j * LANES, LANES)] = jnp.zeros((LANES,), jnp.float32)
            return carry

        lax.fori_loop(0, ZR, fill_z, 0)

        def fill_o(i, carry):
            for j in range(H // LANES):
                ones_v[i, pl.ds(j * LANES, LANES)] = jnp.ones((LANES,), jnp.float32)
            return carry

        lax.fori_loop(0, K, fill_o, 0)

        pltpu.sync_copy(dst_hbm.at[w], dstp)
        for kblk in range(RPT // ZR):
            pltpu.sync_copy(zbuf, accd.at[pl.ds(s * RPT + kblk * ZR, ZR)])
        plsc.subcore_barrier()

        def chunk(i, carry):
            pltpu.sync_copy(ones_v, accd.at[dstp.at[i]], add=True)
            return carry

        lax.fori_loop(0, NCH, chunk, 0)

        plsc.subcore_barrier()
        for kblk in range(RPT // ZR):
            pltpu.sync_copy(accd.at[pl.ds(s * RPT + kblk * ZR, ZR)], zbuf)
            pltpu.sync_copy(zbuf, out_hbm.at[c, pl.ds(s * RPT + kblk * ZR, ZR)])

    return deg_kernel(dst)


def _scatter_sc(hs, src, dst):
    @functools.partial(
        pl.kernel,
        out_type=jax.ShapeDtypeStruct((NC, NP, H), jnp.float32),
        mesh=_sc_mesh(),
        scratch_types=[
            pltpu.VMEM_SHARED((NP, H), jnp.float32),
            pltpu.VMEM((K,), jnp.int32),
            pltpu.VMEM((K,), jnp.int32),
            pltpu.VMEM((K,), jnp.int32),
            pltpu.VMEM((K,), jnp.int32),
            pltpu.VMEM((K, H), jnp.float32),
            pltpu.VMEM((K, H), jnp.float32),
            pltpu.VMEM((ZR, H), jnp.float32),
            pltpu.SemaphoreType.DMA,
            pltpu.SemaphoreType.DMA,
            pltpu.SemaphoreType.DMA,
            pltpu.SemaphoreType.DMA,
        ],
    )
    def s_kernel(hs_hbm, src_hbm, dst_hbm, out_hbm, acc, src0, dst0, src1, dst1,
                 rows0, rows1, zbuf, g0, g1, ia, ib):
        c = lax.axis_index("c")
        s = lax.axis_index("s")
        w = c * NS + s

        def fill_z(i, carry):
            for j in range(H // LANES):
                zbuf[i, pl.ds(j * LANES, LANES)] = jnp.zeros((LANES,), jnp.float32)
            return carry

        lax.fori_loop(0, ZR, fill_z, 0)

        for kblk in range(RPT // ZR):
            pltpu.sync_copy(zbuf, acc.at[pl.ds(s * RPT + kblk * ZR, ZR)])
        plsc.subcore_barrier()

        def wait_rows(sem):
            pltpu.make_async_copy(hs_hbm.at[pl.ds(0, K)], rows0, sem).wait()

        def wait_idx(sem):
            pltpu.make_async_copy(src_hbm.at[pl.ds(0, K)], src0, sem).wait()
            pltpu.make_async_copy(src_hbm.at[pl.ds(0, K)], src0, sem).wait()

        base0 = w * EPT
        pltpu.sync_copy(src_hbm.at[pl.ds(base0, K)], src0)
        pltpu.sync_copy(dst_hbm.at[pl.ds(base0, K)], dst0)
        pltpu.async_copy(hs_hbm.at[src0], rows0, g0)
        pltpu.async_copy(src_hbm.at[pl.ds(base0 + K, K)], src1, ib)
        pltpu.async_copy(dst_hbm.at[pl.ds(base0 + K, K)], dst1, ib)

        def pair(j, carry):
            i0 = 2 * j
            b2 = base0 + (i0 + 2) * K
            b3 = base0 + (i0 + 3) * K
            wait_idx(ib)
            wait_rows(g0)
            pltpu.async_copy(hs_hbm.at[src1], rows1, g1)
            pltpu.sync_copy(rows0, acc.at[dst0], add=True)
            pltpu.async_copy(src_hbm.at[pl.ds(b2, K)], src0, ia)
            pltpu.async_copy(dst_hbm.at[pl.ds(b2, K)], dst0, ia)
            wait_idx(ia)
            wait_rows(g1)
            pltpu.async_copy(hs_hbm.at[src0], rows0, g0)
            pltpu.sync_copy(rows1, acc.at[dst1], add=True)
            pltpu.async_copy(src_hbm.at[pl.ds(b3, K)], src1, ib)
            pltpu.async_copy(dst_hbm.at[pl.ds(b3, K)], dst1, ib)
            return carry

        lax.fori_loop(0, NCH // 2 - 1, pair, 0)

        wait_idx(ib)
        wait_rows(g0)
        pltpu.async_copy(hs_hbm.at[src1], rows1, g1)
        pltpu.sync_copy(rows0, acc.at[dst0], add=True)
        wait_rows(g1)
        pltpu.sync_copy(rows1, acc.at[dst1], add=True)

        plsc.subcore_barrier()
        for kblk in range(RPT // ZR):
            pltpu.sync_copy(acc.at[pl.ds(s * RPT + kblk * ZR, ZR)], zbuf)
            pltpu.sync_copy(zbuf, out_hbm.at[c, pl.ds(s * RPT + kblk * ZR, ZR)])

    return s_kernel(hs, src, dst)


def _dinv_tc(degp):
    def body(deg_ref, o_ref):
        d = deg_ref[0, :, 0:1] + deg_ref[1, :, 0:1] + 1.0
        o_ref[...] = jnp.broadcast_to(lax.rsqrt(d), (BM, H))

    return pl.pallas_call(
        body,
        grid=(N // BM,),
        in_specs=[pl.BlockSpec((NC, BM, H), lambda i: (0, i, 0))],
        out_specs=pl.BlockSpec((BM, H), lambda i: (i, 0)),
        out_shape=jax.ShapeDtypeStruct((N, H), jnp.float32),
    )(degp)


def _mm_tc(x, W, dinv):
    def body(x_ref, w_ref, dinv_ref, o_ref):
        h = jnp.dot(x_ref[...], w_ref[...],
                    preferred_element_type=jnp.float32,
                    precision=lax.Precision.HIGHEST)
        o_ref[...] = h * dinv_ref[...]

    return pl.pallas_call(
        body,
        grid=(N // BM,),
        in_specs=[
            pl.BlockSpec((BM, H), lambda i: (i, 0)),
            pl.BlockSpec((H, H), lambda i: (0, 0)),
            pl.BlockSpec((BM, H), lambda i: (i, 0)),
        ],
        out_specs=pl.BlockSpec((BM, H), lambda i: (i, 0)),
        out_shape=jax.ShapeDtypeStruct((N, H), jnp.float32),
    )(x, W, dinv)


def _f1_tc(p, hs, dinv, b):
    def body(p_ref, hs_ref, dinv_ref, b_ref, t_ref, st_ref):
        i = pl.program_id(0)
        t = (p_ref[0] + p_ref[1] + hs_ref[...]) * dinv_ref[...] + b_ref[...]
        t_ref[...] = t

        @pl.when(i == 0)
        def _():
            st_ref[...] = jnp.zeros((8, H), jnp.float32)

        st_ref[0:1, :] += jnp.sum(t, axis=0, keepdims=True)
        st_ref[1:2, :] += jnp.sum(t * t, axis=0, keepdims=True)

    return pl.pallas_call(
        body,
        grid=(N // BM,),
        in_specs=[
            pl.BlockSpec((NC, BM, H), lambda i: (0, i, 0)),
            pl.BlockSpec((BM, H), lambda i: (i, 0)),
            pl.BlockSpec((BM, H), lambda i: (i, 0)),
            pl.BlockSpec((1, H), lambda i: (0, 0)),
        ],
        out_specs=[
            pl.BlockSpec((BM, H), lambda i: (i, 0)),
            pl.BlockSpec((8, H), lambda i: (0, 0)),
        ],
        out_shape=[
            jax.ShapeDtypeStruct((N, H), jnp.float32),
            jax.ShapeDtypeStruct((8, H), jnp.float32),
        ],
    )(p, hs, dinv, b)


def _f2_tc(t, stats, g, be, x):
    def body(t_ref, st_ref, g_ref, be_ref, x_ref, o_ref):
        m = st_ref[0:1, :] * (1.0 / N)
        ex2 = st_ref[1:2, :] * (1.0 / N)
        v = ex2 - m * m
        scale = lax.rsqrt(v + BN_EPS) * g_ref[...]
        h = (t_ref[...] - m) * scale + be_ref[...]
        o_ref[...] = x_ref[...] + jnp.maximum(h, 0.0)

    return pl.pallas_call(
        body,
        grid=(N // BM,),
        in_specs=[
            pl.BlockSpec((BM, H), lambda i: (i, 0)),
            pl.BlockSpec((8, H), lambda i: (0, 0)),
            pl.BlockSpec((1, H), lambda i: (0, 0)),
            pl.BlockSpec((1, H), lambda i: (0, 0)),
            pl.BlockSpec((BM, H), lambda i: (i, 0)),
        ],
        out_specs=pl.BlockSpec((BM, H), lambda i: (i, 0)),
        out_shape=jax.ShapeDtypeStruct((N, H), jnp.float32),
    )(t, stats, g, be, x)


def kernel(x, edge_index, W0, b0, g0, be0, W1, b1, g1, be1, W2, b2, g2, be2):
    src = edge_index[0]
    dst = edge_index[1]

    degp = _deg_sc(dst.reshape(NW, NCH, K))
    dinv = _dinv_tc(degp)

    for (W, b, g, be) in ((W0, b0, g0, be0), (W1, b1, g1, be1), (W2, b2, g2, be2)):
        hs = _mm_tc(x, W, dinv)
        p = _scatter_sc(hs, src, dst)
        t, stats = _f1_tc(p, hs, dinv, b.reshape(1, H))
        x = _f2_tc(t, stats, g.reshape(1, H), be.reshape(1, H), x)
    return x


# speedup vs baseline: 20.0364x; 1.5554x over previous
import functools

import jax
import jax.numpy as jnp
from jax import lax
from jax.experimental import pallas as pl
from jax.experimental.pallas import tpu as pltpu
from jax.experimental.pallas import tpu_sc as plsc

N = 10000
E = 320000
H = 128
LANES = 16
NC = 2
NS = 16
NW = NC * NS
KE = 128
NCHP = 80
EPT_P = NCHP * KE
E_P = NW * EPT_P
PAD_PT = EPT_P - E // NW
NP = 10240
RPT = NP // NS
ZR = 64
BN_EPS = 1e-5
BM = 2000
BMM = 2048


def _sc_mesh():
    return plsc.VectorSubcoreMesh(core_axis_name="c", subcore_axis_name="s")


def _deg_sc(edges):
    @functools.partial(
        pl.kernel,
        out_type=jax.ShapeDtypeStruct((NC, NP, H), jnp.float32),
        mesh=_sc_mesh(),
        scratch_types=[
            pltpu.VMEM_SHARED((NP, H), jnp.float32),
            pltpu.VMEM((2, KE), jnp.int32),
            pltpu.VMEM((2, KE), jnp.int32),
            pltpu.VMEM((KE, H), jnp.float32),
            pltpu.VMEM((ZR, H), jnp.float32),
            pltpu.SemaphoreType.DMA,
            pltpu.SemaphoreType.DMA,
        ],
    )
    def deg_kernel(edge_hbm, out_hbm, accd, e0, e1, ones_v, zbuf, ia, ib):
        c = lax.axis_index("c")
        s = lax.axis_index("s")
        w = c * NS + s
        base0 = w * EPT_P

        def fill_z(i, carry):
            for j in range(H // LANES):
                zbuf[i, pl.ds(j * LANES, LANES)] = jnp.zeros((LANES,), jnp.float32)
            return carry

        lax.fori_loop(0, ZR, fill_z, 0)

        def fill_o(i, carry):
            for j in range(H // LANES):
                ones_v[i, pl.ds(j * LANES, LANES)] = jnp.ones((LANES,), jnp.float32)
            return carry

        lax.fori_loop(0, KE, fill_o, 0)

        for kblk in range(RPT // ZR):
            pltpu.sync_copy(zbuf, accd.at[pl.ds(s * RPT + kblk * ZR, ZR)])
        plsc.subcore_barrier()

        def wait_idx(sem, buf):
            pltpu.make_async_copy(edge_hbm.at[:, pl.ds(0, KE)], buf, sem).wait()

        pltpu.sync_copy(edge_hbm.at[:, pl.ds(base0, KE)], e0)
        pltpu.async_copy(edge_hbm.at[:, pl.ds(base0 + KE, KE)], e1, ib)

        def pair(j, carry):
            i0 = 2 * j
            pltpu.sync_copy(ones_v, accd.at[e0.at[1]], add=True)
            pltpu.async_copy(edge_hbm.at[:, pl.ds(base0 + (i0 + 2) * KE, KE)], e0, ia)
            wait_idx(ib, e1)
            pltpu.sync_copy(ones_v, accd.at[e1.at[1]], add=True)
            pltpu.async_copy(edge_hbm.at[:, pl.ds(base0 + (i0 + 3) * KE, KE)], e1, ib)
            wait_idx(ia, e0)
            return carry

        lax.fori_loop(0, NCHP // 2 - 1, pair, 0)

        pltpu.sync_copy(ones_v, accd.at[e0.at[1]], add=True)
        wait_idx(ib, e1)
        pltpu.sync_copy(ones_v, accd.at[e1.at[1]], add=True)

        plsc.subcore_barrier()
        for kblk in range(RPT // ZR):
            pltpu.sync_copy(accd.at[pl.ds(s * RPT + kblk * ZR, ZR)], zbuf)
            pltpu.sync_copy(zbuf, out_hbm.at[c, pl.ds(s * RPT + kblk * ZR, ZR)])

    return deg_kernel(edges)


def _scatter_sc(hs, edges):
    @functools.partial(
        pl.kernel,
        out_type=jax.ShapeDtypeStruct((NC, NP, H), jnp.float32),
        mesh=_sc_mesh(),
        scratch_types=[
            pltpu.VMEM_SHARED((NP, H), jnp.float32),
            pltpu.VMEM((2, KE), jnp.int32),
            pltpu.VMEM((2, KE), jnp.int32),
            pltpu.VMEM((KE, H), jnp.float32),
            pltpu.VMEM((KE, H), jnp.float32),
            pltpu.VMEM((ZR, H), jnp.float32),
            pltpu.SemaphoreType.DMA,
            pltpu.SemaphoreType.DMA,
            pltpu.SemaphoreType.DMA,
            pltpu.SemaphoreType.DMA,
        ],
    )
    def s_kernel(hs_hbm, edge_hbm, out_hbm, acc, e0, e1,
                 rows0, rows1, zbuf, g0, g1, ia, ib):
        c = lax.axis_index("c")
        s = lax.axis_index("s")
        w = c * NS + s
        base0 = w * EPT_P

        def fill_z(i, carry):
            for j in range(H // LANES):
                zbuf[i, pl.ds(j * LANES, LANES)] = jnp.zeros((LANES,), jnp.float32)
            return carry

        lax.fori_loop(0, ZR, fill_z, 0)

        for kblk in range(RPT // ZR):
            pltpu.sync_copy(zbuf, acc.at[pl.ds(s * RPT + kblk * ZR, ZR)])
        plsc.subcore_barrier()

        def wait_rows(sem):
            pltpu.make_async_copy(hs_hbm.at[pl.ds(0, KE)], rows0, sem).wait()

        def wait_idx(sem):
            pltpu.make_async_copy(edge_hbm.at[:, pl.ds(0, KE)], e0, sem).wait()

        pltpu.sync_copy(edge_hbm.at[:, pl.ds(base0, KE)], e0)
        pltpu.async_copy(hs_hbm.at[e0.at[0]], rows0, g0)
        pltpu.async_copy(edge_hbm.at[:, pl.ds(base0 + KE, KE)], e1, ib)

        def pair(j, carry):
            i0 = 2 * j
            wait_idx(ib)
            wait_rows(g0)
            pltpu.async_copy(hs_hbm.at[e1.at[0]], rows1, g1)
            pltpu.sync_copy(rows0, acc.at[e0.at[1]], add=True)
            pltpu.async_copy(edge_hbm.at[:, pl.ds(base0 + (i0 + 2) * KE, KE)], e0, ia)
            wait_idx(ia)
            wait_rows(g1)
            pltpu.async_copy(hs_hbm.at[e0.at[0]], rows0, g0)
            pltpu.sync_copy(rows1, acc.at[e1.at[1]], add=True)
            pltpu.async_copy(edge_hbm.at[:, pl.ds(base0 + (i0 + 3) * KE, KE)], e1, ib)
            return carry

        lax.fori_loop(0, NCHP // 2 - 1, pair, 0)

        wait_idx(ib)
        wait_rows(g0)
        pltpu.async_copy(hs_hbm.at[e1.at[0]], rows1, g1)
        pltpu.sync_copy(rows0, acc.at[e0.at[1]], add=True)
        wait_rows(g1)
        pltpu.sync_copy(rows1, acc.at[e1.at[1]], add=True)

        plsc.subcore_barrier()
        for kblk in range(RPT // ZR):
            pltpu.sync_copy(acc.at[pl.ds(s * RPT + kblk * ZR, ZR)], zbuf)
            pltpu.sync_copy(zbuf, out_hbm.at[c, pl.ds(s * RPT + kblk * ZR, ZR)])

    return s_kernel(hs, edges)


def _dinv_tc(degp):
    def body(deg_ref, o_ref):
        d = deg_ref[0, :, 0:1] + deg_ref[1, :, 0:1] + 1.0
        o_ref[...] = jnp.broadcast_to(lax.rsqrt(d), (BMM, H))

    return pl.pallas_call(
        body,
        grid=(NP // BMM,),
        in_specs=[pl.BlockSpec((NC, BMM, H), lambda i: (0, i, 0))],
        out_specs=pl.BlockSpec((BMM, H), lambda i: (i, 0)),
        out_shape=jax.ShapeDtypeStruct((NP, H), jnp.float32),
    )(degp)


def _mm_tc(x, W, dinv):
    def body(x_ref, w_ref, dinv_ref, o_ref):
        h = jnp.dot(x_ref[...], w_ref[...],
                    preferred_element_type=jnp.float32,
                    precision=lax.Precision.HIGHEST)
        o_ref[...] = h * dinv_ref[...]

    return pl.pallas_call(
        body,
        grid=(NP // BMM,),
        in_specs=[
            pl.BlockSpec((BMM, H), lambda i: (i, 0)),
            pl.BlockSpec((H, H), lambda i: (0, 0)),
            pl.BlockSpec((BMM, H), lambda i: (i, 0)),
        ],
        out_specs=pl.BlockSpec((BMM, H), lambda i: (i, 0)),
        out_shape=jax.ShapeDtypeStruct((NP, H), jnp.float32),
    )(x, W, dinv)


def _f1_tc(p, hs, dinv, b):
    def body(p_ref, hs_ref, dinv_ref, b_ref, t_ref, st_ref):
        i = pl.program_id(0)
        t = (p_ref[0] + p_ref[1] + hs_ref[...]) * dinv_ref[...] + b_ref[...]
        t_ref[...] = t

        @pl.when(i == 0)
        def _():
            st_ref[...] = jnp.zeros((8, H), jnp.float32)

        st_ref[0:1, :] += jnp.sum(t, axis=0, keepdims=True)
        st_ref[1:2, :] += jnp.sum(t * t, axis=0, keepdims=True)

    return pl.pallas_call(
        body,
        grid=(N // BM,),
        in_specs=[
            pl.BlockSpec((NC, BM, H), lambda i: (0, i, 0)),
            pl.BlockSpec((BM, H), lambda i: (i, 0)),
            pl.BlockSpec((BM, H), lambda i: (i, 0)),
            pl.BlockSpec((1, H), lambda i: (0, 0)),
        ],
        out_specs=[
            pl.BlockSpec((BM, H), lambda i: (i, 0)),
            pl.BlockSpec((8, H), lambda i: (0, 0)),
        ],
        out_shape=[
            jax.ShapeDtypeStruct((N, H), jnp.float32),
            jax.ShapeDtypeStruct((8, H), jnp.float32),
        ],
    )(p, hs, dinv, b)


def _f2_tc(t, stats, g, be, x):
    def body(t_ref, st_ref, g_ref, be_ref, x_ref, o_ref):
        m = st_ref[0:1, :] * (1.0 / N)
        ex2 = st_ref[1:2, :] * (1.0 / N)
        v = ex2 - m * m
        scale = lax.rsqrt(v + BN_EPS) * g_ref[...]
        h = (t_ref[...] - m) * scale + be_ref[...]
        o_ref[...] = x_ref[...] + jnp.maximum(h, 0.0)

    return pl.pallas_call(
        body,
        grid=(N // BM,),
        in_specs=[
            pl.BlockSpec((BM, H), lambda i: (i, 0)),
            pl.BlockSpec((8, H), lambda i: (0, 0)),
            pl.BlockSpec((1, H), lambda i: (0, 0)),
            pl.BlockSpec((1, H), lambda i: (0, 0)),
            pl.BlockSpec((BM, H), lambda i: (i, 0)),
        ],
        out_specs=pl.BlockSpec((BM, H), lambda i: (i, 0)),
        out_shape=jax.ShapeDtypeStruct((N, H), jnp.float32),
    )(t, stats, g, be, x)


def kernel(x, edge_index, W0, b0, g0, be0, W1, b1, g1, be1, W2, b2, g2, be2):
    pad_rows = (jnp.arange(PAD_PT, dtype=jnp.int32) % 128) + N
    pad_blk = jnp.broadcast_to(pad_rows, (2, NW, PAD_PT))
    e3 = jnp.concatenate([edge_index.reshape(2, NW, E // NW), pad_blk], axis=2)
    edges = e3.reshape(2, E_P)

    degp = _deg_sc(edges)
    dinv = _dinv_tc(degp)

    for (W, b, g, be) in ((W0, b0, g0, be0), (W1, b1, g1, be1), (W2, b2, g2, be2)):
        hs = _mm_tc(x, W, dinv)
        p = _scatter_sc(hs, edges)
        t, stats = _f1_tc(p, hs, dinv, b.reshape(1, H))
        x = _f2_tc(t, stats, g.reshape(1, H), be.reshape(1, H), x)
    return x


# speedup vs baseline: 20.2594x; 1.0111x over previous
import functools

import jax
import jax.numpy as jnp
from jax import lax
from jax.experimental import pallas as pl
from jax.experimental.pallas import tpu as pltpu
from jax.experimental.pallas import tpu_sc as plsc

N = 10000
E = 320000
H = 128
LANES = 16
NC = 2
NS = 16
NW = NC * NS
KE = 128
NCHP = 80
EPT_P = NCHP * KE
E_P = NW * EPT_P
PAD_PT = EPT_P - E // NW
NP = 10240
RPT = NP // NS
ZR = 64
BN_EPS = 1e-5
BM = 2000
BMM = 2048


def _sc_mesh():
    return plsc.VectorSubcoreMesh(core_axis_name="c", subcore_axis_name="s")


def _deg_sc(edges):
    @functools.partial(
        pl.kernel,
        out_type=jax.ShapeDtypeStruct((NC, NP, H), jnp.float32),
        mesh=_sc_mesh(),
        scratch_types=[
            pltpu.VMEM_SHARED((NP, H), jnp.float32),
            pltpu.VMEM((2, KE), jnp.int32),
            pltpu.VMEM((2, KE), jnp.int32),
            pltpu.VMEM((KE, H), jnp.float32),
            pltpu.VMEM((ZR, H), jnp.float32),
            pltpu.SemaphoreType.DMA,
            pltpu.SemaphoreType.DMA,
        ],
    )
    def deg_kernel(edge_hbm, out_hbm, accd, e0, e1, ones_v, zbuf, ia, ib):
        c = lax.axis_index("c")
        s = lax.axis_index("s")
        w = c * NS + s
        base0 = w * EPT_P

        def fill_z(i, carry):
            for j in range(H // LANES):
                zbuf[i, pl.ds(j * LANES, LANES)] = jnp.zeros((LANES,), jnp.float32)
            return carry

        lax.fori_loop(0, ZR, fill_z, 0)

        def fill_o(i, carry):
            for j in range(H // LANES):
                ones_v[i, pl.ds(j * LANES, LANES)] = jnp.ones((LANES,), jnp.float32)
            return carry

        lax.fori_loop(0, KE, fill_o, 0)

        for kblk in range(RPT // ZR):
            pltpu.sync_copy(zbuf, accd.at[pl.ds(s * RPT + kblk * ZR, ZR)])
        plsc.subcore_barrier()

        def wait_idx(sem, buf):
            pltpu.make_async_copy(edge_hbm.at[:, pl.ds(0, KE)], buf, sem).wait()

        pltpu.sync_copy(edge_hbm.at[:, pl.ds(base0, KE)], e0)
        pltpu.async_copy(edge_hbm.at[:, pl.ds(base0 + KE, KE)], e1, ib)

        def pair(j, carry):
            i0 = 2 * j
            pltpu.sync_copy(ones_v, accd.at[e0.at[1]], add=True)
            pltpu.async_copy(edge_hbm.at[:, pl.ds(base0 + (i0 + 2) * KE, KE)], e0, ia)
            wait_idx(ib, e1)
            pltpu.sync_copy(ones_v, accd.at[e1.at[1]], add=True)
            pltpu.async_copy(edge_hbm.at[:, pl.ds(base0 + (i0 + 3) * KE, KE)], e1, ib)
            wait_idx(ia, e0)
            return carry

        lax.fori_loop(0, NCHP // 2 - 1, pair, 0)

        pltpu.sync_copy(ones_v, accd.at[e0.at[1]], add=True)
        wait_idx(ib, e1)
        pltpu.sync_copy(ones_v, accd.at[e1.at[1]], add=True)

        plsc.subcore_barrier()
        pltpu.sync_copy(accd.at[pl.ds(s * RPT, RPT)], out_hbm.at[c, pl.ds(s * RPT, RPT)])

    return deg_kernel(edges)


def _scatter_sc(hs, edges):
    @functools.partial(
        pl.kernel,
        out_type=jax.ShapeDtypeStruct((NC, NP, H), jnp.float32),
        mesh=_sc_mesh(),
        scratch_types=[
            pltpu.VMEM_SHARED((NP, H), jnp.float32),
            pltpu.VMEM((2, KE), jnp.int32),
            pltpu.VMEM((2, KE), jnp.int32),
            pltpu.VMEM((KE, H), jnp.float32),
            pltpu.VMEM((KE, H), jnp.float32),
            pltpu.VMEM((ZR, H), jnp.float32),
            pltpu.SemaphoreType.DMA,
            pltpu.SemaphoreType.DMA,
            pltpu.SemaphoreType.DMA,
            pltpu.SemaphoreType.DMA,
        ],
    )
    def s_kernel(hs_hbm, edge_hbm, out_hbm, acc, e0, e1,
                 rows0, rows1, zbuf, g0, g1, ia, ib):
        c = lax.axis_index("c")
        s = lax.axis_index("s")
        w = c * NS + s
        base0 = w * EPT_P

        def fill_z(i, carry):
            for j in range(H // LANES):
                zbuf[i, pl.ds(j * LANES, LANES)] = jnp.zeros((LANES,), jnp.float32)
            return carry

        lax.fori_loop(0, ZR, fill_z, 0)

        for kblk in range(RPT // ZR):
            pltpu.sync_copy(zbuf, acc.at[pl.ds(s * RPT + kblk * ZR, ZR)])
        plsc.subcore_barrier()

        def wait_rows(sem):
            pltpu.make_async_copy(hs_hbm.at[pl.ds(0, KE)], rows0, sem).wait()

        def wait_idx(sem):
            pltpu.make_async_copy(edge_hbm.at[:, pl.ds(0, KE)], e0, sem).wait()

        pltpu.sync_copy(edge_hbm.at[:, pl.ds(base0, KE)], e0)
        pltpu.async_copy(hs_hbm.at[e0.at[0]], rows0, g0)
        pltpu.async_copy(edge_hbm.at[:, pl.ds(base0 + KE, KE)], e1, ib)

        def pair(j, carry):
            i0 = 2 * j
            wait_idx(ib)
            wait_rows(g0)
            pltpu.async_copy(hs_hbm.at[e1.at[0]], rows1, g1)
            pltpu.sync_copy(rows0, acc.at[e0.at[1]], add=True)
            pltpu.async_copy(edge_hbm.at[:, pl.ds(base0 + (i0 + 2) * KE, KE)], e0, ia)
            wait_idx(ia)
            wait_rows(g1)
            pltpu.async_copy(hs_hbm.at[e0.at[0]], rows0, g0)
            pltpu.sync_copy(rows1, acc.at[e1.at[1]], add=True)
            pltpu.async_copy(edge_hbm.at[:, pl.ds(base0 + (i0 + 3) * KE, KE)], e1, ib)
            return carry

        lax.fori_loop(0, NCHP // 2 - 1, pair, 0)

        wait_idx(ib)
        wait_rows(g0)
        pltpu.async_copy(hs_hbm.at[e1.at[0]], rows1, g1)
        pltpu.sync_copy(rows0, acc.at[e0.at[1]], add=True)
        wait_rows(g1)
        pltpu.sync_copy(rows1, acc.at[e1.at[1]], add=True)

        plsc.subcore_barrier()
        pltpu.sync_copy(acc.at[pl.ds(s * RPT, RPT)], out_hbm.at[c, pl.ds(s * RPT, RPT)])

    return s_kernel(hs, edges)


def _dinv_tc(degp):
    def body(deg_ref, o_ref):
        d = deg_ref[0, :, 0:1] + deg_ref[1, :, 0:1] + 1.0
        o_ref[...] = jnp.broadcast_to(lax.rsqrt(d), (BMM, H))

    return pl.pallas_call(
        body,
        grid=(NP // BMM,),
        in_specs=[pl.BlockSpec((NC, BMM, H), lambda i: (0, i, 0))],
        out_specs=pl.BlockSpec((BMM, H), lambda i: (i, 0)),
        out_shape=jax.ShapeDtypeStruct((NP, H), jnp.float32),
    )(degp)


def _mm_tc(x, W, dinv):
    def body(x_ref, w_ref, dinv_ref, o_ref):
        h = jnp.dot(x_ref[...], w_ref[...],
                    preferred_element_type=jnp.float32,
                    precision=lax.Precision.HIGHEST)
        o_ref[...] = h * dinv_ref[...]

    return pl.pallas_call(
        body,
        grid=(NP // BMM,),
        in_specs=[
            pl.BlockSpec((BMM, H), lambda i: (i, 0)),
            pl.BlockSpec((H, H), lambda i: (0, 0)),
            pl.BlockSpec((BMM, H), lambda i: (i, 0)),
        ],
        out_specs=pl.BlockSpec((BMM, H), lambda i: (i, 0)),
        out_shape=jax.ShapeDtypeStruct((NP, H), jnp.float32),
    )(x, W, dinv)


def _f1_tc(p, hs, dinv, b):
    def body(p_ref, hs_ref, dinv_ref, b_ref, t_ref, st_ref):
        i = pl.program_id(0)
        t = (p_ref[0] + p_ref[1] + hs_ref[...]) * dinv_ref[...] + b_ref[...]
        t_ref[...] = t

        @pl.when(i == 0)
        def _():
            st_ref[...] = jnp.zeros((8, H), jnp.float32)

        st_ref[0:1, :] += jnp.sum(t, axis=0, keepdims=True)
        st_ref[1:2, :] += jnp.sum(t * t, axis=0, keepdims=True)

    return pl.pallas_call(
        body,
        grid=(N // BM,),
        in_specs=[
            pl.BlockSpec((NC, BM, H), lambda i: (0, i, 0)),
            pl.BlockSpec((BM, H), lambda i: (i, 0)),
            pl.BlockSpec((BM, H), lambda i: (i, 0)),
            pl.BlockSpec((1, H), lambda i: (0, 0)),
        ],
        out_specs=[
            pl.BlockSpec((BM, H), lambda i: (i, 0)),
            pl.BlockSpec((8, H), lambda i: (0, 0)),
        ],
        out_shape=[
            jax.ShapeDtypeStruct((N, H), jnp.float32),
            jax.ShapeDtypeStruct((8, H), jnp.float32),
        ],
    )(p, hs, dinv, b)


def _f2_tc(t, stats, g, be, x):
    def body(t_ref, st_ref, g_ref, be_ref, x_ref, o_ref):
        m = st_ref[0:1, :] * (1.0 / N)
        ex2 = st_ref[1:2, :] * (1.0 / N)
        v = ex2 - m * m
        scale = lax.rsqrt(v + BN_EPS) * g_ref[...]
        h = (t_ref[...] - m) * scale + be_ref[...]
        o_ref[...] = x_ref[...] + jnp.maximum(h, 0.0)

    return pl.pallas_call(
        body,
        grid=(N // BM,),
        in_specs=[
            pl.BlockSpec((BM, H), lambda i: (i, 0)),
            pl.BlockSpec((8, H), lambda i: (0, 0)),
            pl.BlockSpec((1, H), lambda i: (0, 0)),
            pl.BlockSpec((1, H), lambda i: (0, 0)),
            pl.BlockSpec((BM, H), lambda i: (i, 0)),
        ],
        out_specs=pl.BlockSpec((BM, H), lambda i: (i, 0)),
        out_shape=jax.ShapeDtypeStruct((N, H), jnp.float32),
    )(t, stats, g, be, x)


def kernel(x, edge_index, W0, b0, g0, be0, W1, b1, g1, be1, W2, b2, g2, be2):
    pad_rows = (jnp.arange(PAD_PT, dtype=jnp.int32) % 128) + N
    pad_blk = jnp.broadcast_to(pad_rows, (2, NW, PAD_PT))
    e3 = jnp.concatenate([edge_index.reshape(2, NW, E // NW), pad_blk], axis=2)
    edges = e3.reshape(2, E_P)

    degp = _deg_sc(edges)
    dinv = _dinv_tc(degp)

    for (W, b, g, be) in ((W0, b0, g0, be0), (W1, b1, g1, be1), (W2, b2, g2, be2)):
        hs = _mm_tc(x, W, dinv)
        p = _scatter_sc(hs, edges)
        t, stats = _f1_tc(p, hs, dinv, b.reshape(1, H))
        x = _f2_tc(t, stats, g.reshape(1, H), be.reshape(1, H), x)
    return x


# speedup vs baseline: 20.5898x; 1.0163x over previous
import functools

import jax
import jax.numpy as jnp
from jax import lax
from jax.experimental import pallas as pl
from jax.experimental.pallas import tpu as pltpu
from jax.experimental.pallas import tpu_sc as plsc

N = 10000
E = 320000
H = 128
LANES = 16
NC = 2
NS = 16
NW = NC * NS
KE = 128
NCHP = 80
EPT_P = NCHP * KE
E_P = NW * EPT_P
PAD_PT = EPT_P - E // NW
NP = 10240
RPT = NP // NS
ZR = 64
DW = 128
BN_EPS = 1e-5
BM = 2000
BMM = 2048


def _sc_mesh():
    return plsc.VectorSubcoreMesh(core_axis_name="c", subcore_axis_name="s")


def _deg_sc(edges):
    @functools.partial(
        pl.kernel,
        out_type=jax.ShapeDtypeStruct((NC, NP, DW), jnp.float32),
        mesh=_sc_mesh(),
        scratch_types=[
            pltpu.VMEM_SHARED((NP, DW), jnp.float32),
            pltpu.VMEM((2, KE), jnp.int32),
            pltpu.VMEM((2, KE), jnp.int32),
            pltpu.VMEM((KE, DW), jnp.float32),
            pltpu.VMEM((ZR, DW), jnp.float32),
            pltpu.SemaphoreType.DMA,
            pltpu.SemaphoreType.DMA,
        ],
    )
    def deg_kernel(edge_hbm, out_hbm, accd, e0, e1, ones_v, zbuf, ia, ib):
        c = lax.axis_index("c")
        s = lax.axis_index("s")
        w = c * NS + s
        base0 = w * EPT_P

        def fill_z(i, carry):
            for j in range(DW // LANES):
                zbuf[i, pl.ds(j * LANES, LANES)] = jnp.zeros((LANES,), jnp.float32)
            return carry

        lax.fori_loop(0, ZR, fill_z, 0)

        def fill_o(i, carry):
            for j in range(DW // LANES):
                ones_v[i, pl.ds(j * LANES, LANES)] = jnp.ones((LANES,), jnp.float32)
            return carry

        lax.fori_loop(0, KE, fill_o, 0)

        for kblk in range(RPT // ZR):
            pltpu.sync_copy(zbuf, accd.at[pl.ds(s * RPT + kblk * ZR, ZR)])
        plsc.subcore_barrier()

        def wait_idx(sem, buf):
            pltpu.make_async_copy(edge_hbm.at[:, pl.ds(0, KE)], buf, sem).wait()

        pltpu.sync_copy(edge_hbm.at[:, pl.ds(base0, KE)], e0)
        pltpu.async_copy(edge_hbm.at[:, pl.ds(base0 + KE, KE)], e1, ib)

        def pair(j, carry):
            i0 = 2 * j
            pltpu.sync_copy(ones_v, accd.at[e0.at[1]], add=True)
            pltpu.async_copy(edge_hbm.at[:, pl.ds(base0 + (i0 + 2) * KE, KE)], e0, ia)
            wait_idx(ib, e1)
            pltpu.sync_copy(ones_v, accd.at[e1.at[1]], add=True)
            pltpu.async_copy(edge_hbm.at[:, pl.ds(base0 + (i0 + 3) * KE, KE)], e1, ib)
            wait_idx(ia, e0)
            return carry

        lax.fori_loop(0, NCHP // 2 - 1, pair, 0)

        pltpu.sync_copy(ones_v, accd.at[e0.at[1]], add=True)
        wait_idx(ib, e1)
        pltpu.sync_copy(ones_v, accd.at[e1.at[1]], add=True)

        plsc.subcore_barrier()
        pltpu.sync_copy(accd.at[pl.ds(s * RPT, RPT)], out_hbm.at[c, pl.ds(s * RPT, RPT)])

    return deg_kernel(edges)


def _scatter_sc(hs, edges):
    @functools.partial(
        pl.kernel,
        out_type=jax.ShapeDtypeStruct((NC, NP, H), jnp.float32),
        mesh=_sc_mesh(),
        scratch_types=[
            pltpu.VMEM_SHARED((NP, H), jnp.float32),
            pltpu.VMEM((2, KE), jnp.int32),
            pltpu.VMEM((2, KE), jnp.int32),
            pltpu.VMEM((KE, H), jnp.float32),
            pltpu.VMEM((KE, H), jnp.float32),
            pltpu.VMEM((ZR, H), jnp.float32),
            pltpu.SemaphoreType.DMA,
            pltpu.SemaphoreType.DMA,
            pltpu.SemaphoreType.DMA,
            pltpu.SemaphoreType.DMA,
        ],
    )
    def s_kernel(hs_hbm, edge_hbm, out_hbm, acc, e0, e1,
                 rows0, rows1, zbuf, g0, g1, ia, ib):
        c = lax.axis_index("c")
        s = lax.axis_index("s")
        w = c * NS + s
        base0 = w * EPT_P

        def fill_z(i, carry):
            for j in range(H // LANES):
                zbuf[i, pl.ds(j * LANES, LANES)] = jnp.zeros((LANES,), jnp.float32)
            return carry

        lax.fori_loop(0, ZR, fill_z, 0)

        for kblk in range(RPT // ZR):
            pltpu.sync_copy(zbuf, acc.at[pl.ds(s * RPT + kblk * ZR, ZR)])
        plsc.subcore_barrier()

        def wait_rows(sem):
            pltpu.make_async_copy(hs_hbm.at[pl.ds(0, KE)], rows0, sem).wait()

        def wait_idx(sem):
            pltpu.make_async_copy(edge_hbm.at[:, pl.ds(0, KE)], e0, sem).wait()

        pltpu.sync_copy(edge_hbm.at[:, pl.ds(base0, KE)], e0)
        pltpu.async_copy(hs_hbm.at[e0.at[0]], rows0, g0)
        pltpu.async_copy(edge_hbm.at[:, pl.ds(base0 + KE, KE)], e1, ib)

        def pair(j, carry):
            i0 = 2 * j
            wait_idx(ib)
            wait_rows(g0)
            pltpu.async_copy(hs_hbm.at[e1.at[0]], rows1, g1)
            pltpu.sync_copy(rows0, acc.at[e0.at[1]], add=True)
            pltpu.async_copy(edge_hbm.at[:, pl.ds(base0 + (i0 + 2) * KE, KE)], e0, ia)
            wait_idx(ia)
            wait_rows(g1)
            pltpu.async_copy(hs_hbm.at[e0.at[0]], rows0, g0)
            pltpu.sync_copy(rows1, acc.at[e1.at[1]], add=True)
            pltpu.async_copy(edge_hbm.at[:, pl.ds(base0 + (i0 + 3) * KE, KE)], e1, ib)
            return carry

        lax.fori_loop(0, NCHP // 2 - 1, pair, 0)

        wait_idx(ib)
        wait_rows(g0)
        pltpu.async_copy(hs_hbm.at[e1.at[0]], rows1, g1)
        pltpu.sync_copy(rows0, acc.at[e0.at[1]], add=True)
        wait_rows(g1)
        pltpu.sync_copy(rows1, acc.at[e1.at[1]], add=True)

        plsc.subcore_barrier()
        pltpu.sync_copy(acc.at[pl.ds(s * RPT, RPT)], out_hbm.at[c, pl.ds(s * RPT, RPT)])

    return s_kernel(hs, edges)


def _dinv_tc(degp):
    def body(deg_ref, o_ref):
        d = deg_ref[0, :, 0:1] + deg_ref[1, :, 0:1] + 1.0
        o_ref[...] = jnp.broadcast_to(lax.rsqrt(d), (BM, H))

    return pl.pallas_call(
        body,
        grid=(N // BM,),
        in_specs=[pl.BlockSpec((NC, BM, DW), lambda i: (0, i, 0))],
        out_specs=pl.BlockSpec((BM, H), lambda i: (i, 0)),
        out_shape=jax.ShapeDtypeStruct((N, H), jnp.float32),
    )(degp)


def _mm_tc(x, W, dinv):
    def body(x_ref, w_ref, dinv_ref, o_ref):
        h = jnp.dot(x_ref[...], w_ref[...],
                    preferred_element_type=jnp.float32,
                    precision=lax.Precision.HIGHEST)
        o_ref[...] = h * dinv_ref[...]

    return pl.pallas_call(
        body,
        grid=(N // BM,),
        in_specs=[
            pl.BlockSpec((BM, H), lambda i: (i, 0)),
            pl.BlockSpec((H, H), lambda i: (0, 0)),
            pl.BlockSpec((BM, H), lambda i: (i, 0)),
        ],
        out_specs=pl.BlockSpec((BM, H), lambda i: (i, 0)),
        out_shape=jax.ShapeDtypeStruct((N, H), jnp.float32),
    )(x, W, dinv)


def _f1_tc(p, hs, dinv, b):
    def body(p_ref, hs_ref, dinv_ref, b_ref, t_ref, st_ref):
        i = pl.program_id(0)
        t = (p_ref[0] + p_ref[1] + hs_ref[...]) * dinv_ref[...] + b_ref[...]
        t_ref[...] = t

        @pl.when(i == 0)
        def _():
            st_ref[...] = jnp.zeros((8, H), jnp.float32)

        st_ref[0:1, :] += jnp.sum(t, axis=0, keepdims=True)
        st_ref[1:2, :] += jnp.sum(t * t, axis=0, keepdims=True)

    return pl.pallas_call(
        body,
        grid=(N // BM,),
        in_specs=[
            pl.BlockSpec((NC, BM, H), lambda i: (0, i, 0)),
            pl.BlockSpec((BM, H), lambda i: (i, 0)),
            pl.BlockSpec((BM, H), lambda i: (i, 0)),
            pl.BlockSpec((1, H), lambda i: (0, 0)),
        ],
        out_specs=[
            pl.BlockSpec((BM, H), lambda i: (i, 0)),
            pl.BlockSpec((8, H), lambda i: (0, 0)),
        ],
        out_shape=[
            jax.ShapeDtypeStruct((N, H), jnp.float32),
            jax.ShapeDtypeStruct((8, H), jnp.float32),
        ],
    )(p, hs, dinv, b)


def _f2_tc(t, stats, g, be, x):
    def body(t_ref, st_ref, g_ref, be_ref, x_ref, o_ref):
        m = st_ref[0:1, :] * (1.0 / N)
        ex2 = st_ref[1:2, :] * (1.0 / N)
        v = ex2 - m * m
        scale = lax.rsqrt(v + BN_EPS) * g_ref[...]
        h = (t_ref[...] - m) * scale + be_ref[...]
        o_ref[...] = x_ref[...] + jnp.maximum(h, 0.0)

    return pl.pallas_call(
        body,
        grid=(N // BM,),
        in_specs=[
            pl.BlockSpec((BM, H), lambda i: (i, 0)),
            pl.BlockSpec((8, H), lambda i: (0, 0)),
            pl.BlockSpec((1, H), lambda i: (0, 0)),
            pl.BlockSpec((1, H), lambda i: (0, 0)),
            pl.BlockSpec((BM, H), lambda i: (i, 0)),
        ],
        out_specs=pl.BlockSpec((BM, H), lambda i: (i, 0)),
        out_shape=jax.ShapeDtypeStruct((N, H), jnp.float32),
    )(t, stats, g, be, x)


def _f2mm_tc(t, stats, g, be, x, Wn, dinv):
    def body(t_ref, st_ref, g_ref, be_ref, x_ref, w_ref, dinv_ref, o_ref, hs_ref):
        m = st_ref[0:1, :] * (1.0 / N)
        ex2 = st_ref[1:2, :] * (1.0 / N)
        v = ex2 - m * m
        scale = lax.rsqrt(v + BN_EPS) * g_ref[...]
        h = (t_ref[...] - m) * scale + be_ref[...]
        xn = x_ref[...] + jnp.maximum(h, 0.0)
        o_ref[...] = xn
        hn = jnp.dot(xn, w_ref[...],
                     preferred_element_type=jnp.float32,
                     precision=lax.Precision.HIGHEST)
        hs_ref[...] = hn * dinv_ref[...]

    return pl.pallas_call(
        body,
        grid=(N // BM,),
        in_specs=[
            pl.BlockSpec((BM, H), lambda i: (i, 0)),
            pl.BlockSpec((8, H), lambda i: (0, 0)),
            pl.BlockSpec((1, H), lambda i: (0, 0)),
            pl.BlockSpec((1, H), lambda i: (0, 0)),
            pl.BlockSpec((BM, H), lambda i: (i, 0)),
            pl.BlockSpec((H, H), lambda i: (0, 0)),
            pl.BlockSpec((BM, H), lambda i: (i, 0)),
        ],
        out_specs=[
            pl.BlockSpec((BM, H), lambda i: (i, 0)),
            pl.BlockSpec((BM, H), lambda i: (i, 0)),
        ],
        out_shape=[
            jax.ShapeDtypeStruct((N, H), jnp.float32),
            jax.ShapeDtypeStruct((N, H), jnp.float32),
        ],
    )(t, stats, g, be, x, Wn, dinv)


def kernel(x, edge_index, W0, b0, g0, be0, W1, b1, g1, be1, W2, b2, g2, be2):
    pad_src = jnp.arange(PAD_PT, dtype=jnp.int32) % 128
    pad_dst = pad_src + N
    pad_blk = jnp.stack([jnp.broadcast_to(pad_src, (NW, PAD_PT)),
                         jnp.broadcast_to(pad_dst, (NW, PAD_PT))])
    e3 = jnp.concatenate([edge_index.reshape(2, NW, E // NW), pad_blk], axis=2)
    edges = e3.reshape(2, E_P)

    degp = _deg_sc(edges)
    dinv = _dinv_tc(degp)

    params = ((W0, b0, g0, be0), (W1, b1, g1, be1), (W2, b2, g2, be2))
    hs = _mm_tc(x, W0, dinv)
    for li, (W, b, g, be) in enumerate(params):
        p = _scatter_sc(hs, edges)
        t, stats = _f1_tc(p, hs, dinv, b.reshape(1, H))
        if li < 2:
            x, hs = _f2mm_tc(t, stats, g.reshape(1, H), be.reshape(1, H), x,
                             params[li + 1][0], dinv)
        else:
            x = _f2_tc(t, stats, g.reshape(1, H), be.reshape(1, H), x)
    return x
